# baseline (speedup 1.0000x reference)
import jax
import jax.numpy as jnp
import numpy as np
from jax import lax
from jax.experimental import pallas as pl
from jax.experimental.pallas import tpu as pltpu
from jax.experimental.pallas import tpu_sc as plsc

N_POINTS = 262144
N_LEVELS = 16
F_PER_LEVEL = 2
LOG2_T = 19
T = 1 << LOG2_T
MASK = T - 1
BASE_RES = 16
PER_LEVEL_SCALE = 1.3819129

P1 = int(np.uint32(2654435761).view(np.int32))
P2 = int(np.uint32(805459861).view(np.int32))

RES = [float(np.floor(BASE_RES * (PER_LEVEL_SCALE ** l))) for l in range(N_LEVELS)]

NC = 2
NS = 16
LANES = 16
NW = NC * NS
PPW = N_POINTS // NW
B = 512
G = B // LANES
NCH = PPW // B
NF = N_LEVELS * F_PER_LEVEL

CORNERS = [(i, j, k) for i in (0, 1) for j in (0, 1) for k in (0, 1)]

N_DENSE = 3
_R = [int(RES[l]) + 1 for l in range(N_DENSE)]
_GRID_OFF = [0]
for _l in range(N_DENSE):
    _GRID_OFF.append(_GRID_OFF[-1] + _R[_l] ** 3)
GRID_WORDS = _GRID_OFF[N_DENSE]
STAGE_CHUNK = 8 * B
GRID_PAD = ((GRID_WORDS + STAGE_CHUNK - 1) // STAGE_CHUNK) * STAGE_CHUNK
N_STAGE = GRID_PAD // STAGE_CHUNK


def _stage_idx_np() -> np.ndarray:
    out = np.zeros((GRID_PAD,), np.int32)
    pos = 0
    for l in range(N_DENSE):
        r = _R[l]
        x, y, z = np.meshgrid(np.arange(r, dtype=np.uint32),
                              np.arange(r, dtype=np.uint32),
                              np.arange(r, dtype=np.uint32), indexing="ij")
        with np.errstate(over="ignore"):
            h = (x * np.uint32(1)) ^ (y * np.uint32(2654435761)) \
                ^ (z * np.uint32(805459861))
        idx = (h & np.uint32(MASK)).astype(np.int64) + l * T
        out[pos:pos + r ** 3] = idx.reshape(-1).astype(np.int32)
        pos += r ** 3
    return out


STAGE_IDX = _stage_idx_np()


def _body(x0_hbm, x1_hbm, x2_hbm, tab_hbm, sidx_hbm, out_hbm,
          x_v, frac_v, idx_v, rows_v, out_v, grid_v, sem0, sem1):
    wid = lax.axis_index("s") * NC + lax.axis_index("c")
    lane = lax.iota(jnp.int32, 16)
    zeros16 = lane * 0
    sems = (sem0, sem1)

    for p in range(N_STAGE):
        pltpu.sync_copy(sidx_hbm.at[pl.ds(p * STAGE_CHUNK, STAGE_CHUNK)],
                        idx_v.at[p % 2])
        pltpu.async_copy(tab_hbm.at[idx_v.at[p % 2]],
                         grid_v.at[pl.ds(p * STAGE_CHUNK, STAGE_CHUNK)],
                         sems[p % 2]).wait()

    def chunk_body(c, carry):
        base = wid * PPW + c * B
        pltpu.sync_copy(x0_hbm.at[pl.ds(base, B)], x_v.at[0])
        pltpu.sync_copy(x1_hbm.at[pl.ds(base, B)], x_v.at[1])
        pltpu.sync_copy(x2_hbm.at[pl.ds(base, B)], x_v.at[2])

        def hash_level(l, s):
            res = RES[l]

            def hash_body(g, _):
                o = g * LANES
                x0 = x_v[0, pl.ds(o, LANES)] * res
                x1 = x_v[1, pl.ds(o, LANES)] * res
                x2 = x_v[2, pl.ds(o, LANES)] * res
                p0 = x0.astype(jnp.int32)
                p1 = x1.astype(jnp.int32)
                p2 = x2.astype(jnp.int32)
                frac_v[s, 0, pl.ds(o, LANES)] = x0 - p0.astype(jnp.float32)
                frac_v[s, 1, pl.ds(o, LANES)] = x1 - p1.astype(jnp.float32)
                frac_v[s, 2, pl.ds(o, LANES)] = x2 - p2.astype(jnp.float32)
                hx = (p0, p0 + 1)
                hy0 = p1 * P1
                hy = (hy0, hy0 + P1)
                hz0 = p2 * P2
                hz = (hz0, hz0 + P2)
                for ci, (i, j, k) in enumerate(CORNERS):
                    h = (hx[i] ^ hy[j] ^ hz[k]) & MASK
                    idx_v[s, pl.ds(ci * B + o, LANES)] = h + l * T
                return 0

            lax.fori_loop(0, G, hash_body, 0)

        def fire(s):
            return pltpu.async_copy(tab_hbm.at[idx_v.at[s]], rows_v.at[s], sems[s])

        def combine(l, s):
            def comb_body(g, _):
                o = g * LANES
                fx = frac_v[s, 0, pl.ds(o, LANES)]
                fy = frac_v[s, 1, pl.ds(o, LANES)]
                fz = frac_v[s, 2, pl.ds(o, LANES)]
                wx = (1.0 - fx, fx)
                wy = (1.0 - fy, fy)
                wz = (1.0 - fz, fz)
                acc0 = jnp.zeros((16,), jnp.float32)
                acc1 = jnp.zeros((16,), jnp.float32)
                for ci, (i, j, k) in enumerate(CORNERS):
                    w = wx[i] * wy[j] * wz[k]
                    v = rows_v[s, pl.ds(ci * B + o, LANES)]
                    f0 = lax.bitcast_convert_type(v & jnp.int32(-65536), jnp.float32)
                    f1 = lax.bitcast_convert_type(v << 16, jnp.float32)
                    acc0 = acc0 + w * f0
                    acc1 = acc1 + w * f1
                nidx = lane + o
                plsc.store_scatter(out_v, [nidx, zeros16 + (2 * l)], acc0)
                plsc.store_scatter(out_v, [nidx, zeros16 + (2 * l + 1)], acc1)
                return 0

            lax.fori_loop(0, G, comb_body, 0)

        def fused_dense(l):
            res = RES[l]
            r = _R[l]
            goff = _GRID_OFF[l]

            def fd_body(g, _):
                o = g * LANES
                x0 = x_v[0, pl.ds(o, LANES)] * res
                x1 = x_v[1, pl.ds(o, LANES)] * res
                x2 = x_v[2, pl.ds(o, LANES)] * res
                p0 = x0.astype(jnp.int32)
                p1 = x1.astype(jnp.int32)
                p2 = x2.astype(jnp.int32)
                fx = x0 - p0.astype(jnp.float32)
                fy = x1 - p1.astype(jnp.float32)
                fz = x2 - p2.astype(jnp.float32)
                wx = (1.0 - fx, fx)
                wy = (1.0 - fy, fy)
                wz = (1.0 - fz, fz)
                cid = (p0 * r + p1) * r + p2 + goff
                acc0 = jnp.zeros((16,), jnp.float32)
                acc1 = jnp.zeros((16,), jnp.float32)
                for (i, j, k) in CORNERS:
                    w = wx[i] * wy[j] * wz[k]
                    off = i * r * r + j * r + k
                    v = plsc.load_gather(grid_v, [cid + off])
                    f0 = lax.bitcast_convert_type(v & jnp.int32(-65536), jnp.float32)
                    f1 = lax.bitcast_convert_type(v << 16, jnp.float32)
                    acc0 = acc0 + w * f0
                    acc1 = acc1 + w * f1
                nidx = lane + o
                plsc.store_scatter(out_v, [nidx, zeros16 + (2 * l)], acc0)
                plsc.store_scatter(out_v, [nidx, zeros16 + (2 * l + 1)], acc1)
                return 0

            lax.fori_loop(0, G, fd_body, 0)

        hash_level(N_DENSE, 0)
        descs = {0: fire(0)}
        for l in range(N_DENSE):
            fused_dense(l)
        for l in range(N_DENSE, N_LEVELS):
            s = (l - N_DENSE) % 2
            if l + 1 < N_LEVELS:
                ns = (l + 1 - N_DENSE) % 2
                hash_level(l + 1, ns)
                descs[ns] = fire(ns)
            descs[s].wait()
            combine(l, s)

        pltpu.sync_copy(out_v, out_hbm.at[pl.ds(base, B)])
        return carry

    lax.fori_loop(0, NCH, chunk_body, 0)


@jax.jit
def _encode_sc(x0, x1, x2, tab, sidx):
    mesh = plsc.VectorSubcoreMesh(core_axis_name="c", subcore_axis_name="s")
    return pl.kernel(
        _body,
        out_type=jax.ShapeDtypeStruct((N_POINTS, NF), jnp.float32),
        mesh=mesh,
        compiler_params=pltpu.CompilerParams(
            needs_layout_passes=False, use_tc_tiling_on_sc=False
        ),
        scratch_types=[
            pltpu.VMEM((3, B), jnp.float32),
            pltpu.VMEM((2, 3, B), jnp.float32),
            pltpu.VMEM((2, 8 * B), jnp.int32),
            pltpu.VMEM((2, 8 * B), jnp.int32),
            pltpu.VMEM((B, NF), jnp.float32),
            pltpu.VMEM((GRID_PAD,), jnp.int32),
            pltpu.SemaphoreType.DMA,
            pltpu.SemaphoreType.DMA,
        ],
    )(x0, x1, x2, tab, sidx)


def kernel(x, table):
    x = x.astype(jnp.float32)
    b0 = lax.bitcast_convert_type(
        table[:, :, 0].astype(jnp.bfloat16), jnp.uint16).astype(jnp.int32)
    b1 = lax.bitcast_convert_type(
        table[:, :, 1].astype(jnp.bfloat16), jnp.uint16).astype(jnp.int32)
    tab = ((b0 << 16) | b1).reshape(N_LEVELS * T)
    return _encode_sc(x[:, 0], x[:, 1], x[:, 2], tab, jnp.asarray(STAGE_IDX))


# speedup vs baseline: 1.0811x; 1.0811x over previous
import jax
import jax.numpy as jnp
import numpy as np
from jax import lax
from jax.experimental import pallas as pl
from jax.experimental.pallas import tpu as pltpu
from jax.experimental.pallas import tpu_sc as plsc

N_POINTS = 262144
N_LEVELS = 16
F_PER_LEVEL = 2
LOG2_T = 19
T = 1 << LOG2_T
MASK = T - 1
BASE_RES = 16
PER_LEVEL_SCALE = 1.3819129

P1 = int(np.uint32(2654435761).view(np.int32))
P2 = int(np.uint32(805459861).view(np.int32))

RES = [float(np.floor(BASE_RES * (PER_LEVEL_SCALE ** l))) for l in range(N_LEVELS)]

NC = 2
NS = 16
LANES = 16
NW = NC * NS
PPW = N_POINTS // NW
B = 512
G = B // LANES
NCH = PPW // B
NF = N_LEVELS * F_PER_LEVEL

CORNERS = [(i, j, k) for i in (0, 1) for j in (0, 1) for k in (0, 1)]


def _body(x0_hbm, x1_hbm, x2_hbm, tab_hbm, out_hbm,
          x_v, frac_v, idx_v, rows_v, out_v, sem0, sem1, sem2):
    wid = lax.axis_index("s") * NC + lax.axis_index("c")
    lane = lax.iota(jnp.int32, 16)
    zeros16 = lane * 0
    sems = (sem0, sem1, sem2)

    def chunk_body(c, carry):
        base = wid * PPW + c * B
        pltpu.sync_copy(x0_hbm.at[pl.ds(base, B)], x_v.at[0])
        pltpu.sync_copy(x1_hbm.at[pl.ds(base, B)], x_v.at[1])
        pltpu.sync_copy(x2_hbm.at[pl.ds(base, B)], x_v.at[2])

        def hash_level(l, s):
            res = RES[l]

            def hash_body(g, _):
                o = g * LANES
                x0 = x_v[0, pl.ds(o, LANES)] * res
                x1 = x_v[1, pl.ds(o, LANES)] * res
                x2 = x_v[2, pl.ds(o, LANES)] * res
                p0 = x0.astype(jnp.int32)
                p1 = x1.astype(jnp.int32)
                p2 = x2.astype(jnp.int32)
                frac_v[s, 0, pl.ds(o, LANES)] = x0 - p0.astype(jnp.float32)
                frac_v[s, 1, pl.ds(o, LANES)] = x1 - p1.astype(jnp.float32)
                frac_v[s, 2, pl.ds(o, LANES)] = x2 - p2.astype(jnp.float32)
                hx = (p0, p0 + 1)
                hy0 = p1 * P1
                hy = (hy0, hy0 + P1)
                hz0 = p2 * P2
                hz = (hz0, hz0 + P2)
                for ci, (i, j, k) in enumerate(CORNERS):
                    h = (hx[i] ^ hy[j] ^ hz[k]) & MASK
                    idx_v[s, pl.ds(ci * B + o, LANES)] = h + l * T
                return 0

            lax.fori_loop(0, G, hash_body, 0)

        def fire(s):
            return pltpu.async_copy(tab_hbm.at[idx_v.at[s]], rows_v.at[s], sems[s])

        def combine(l, s):
            def comb_body(g, _):
                o = g * LANES
                fx = frac_v[s, 0, pl.ds(o, LANES)]
                fy = frac_v[s, 1, pl.ds(o, LANES)]
                fz = frac_v[s, 2, pl.ds(o, LANES)]
                wx = (1.0 - fx, fx)
                wy = (1.0 - fy, fy)
                wz = (1.0 - fz, fz)
                acc0 = jnp.zeros((16,), jnp.float32)
                acc1 = jnp.zeros((16,), jnp.float32)
                for ci, (i, j, k) in enumerate(CORNERS):
                    w = wx[i] * wy[j] * wz[k]
                    v = rows_v[s, pl.ds(ci * B + o, LANES)]
                    f0 = lax.bitcast_convert_type(v & jnp.int32(-65536), jnp.float32)
                    f1 = lax.bitcast_convert_type(v << 16, jnp.float32)
                    acc0 = acc0 + w * f0
                    acc1 = acc1 + w * f1
                nidx = lane + o
                plsc.store_scatter(out_v, [nidx, zeros16 + (2 * l)], acc0)
                plsc.store_scatter(out_v, [nidx, zeros16 + (2 * l + 1)], acc1)
                return 0

            lax.fori_loop(0, G, comb_body, 0)

        hash_level(0, 0)
        descs = {0: fire(0)}
        hash_level(1, 1)
        descs[1] = fire(1)
        for l in range(N_LEVELS):
            s = l % 3
            if l + 2 < N_LEVELS:
                ns = (l + 2) % 3
                hash_level(l + 2, ns)
                descs[ns] = fire(ns)
            descs[s].wait()
            combine(l, s)

        pltpu.sync_copy(out_v, out_hbm.at[pl.ds(base, B)])
        return carry

    lax.fori_loop(0, NCH, chunk_body, 0)


@jax.jit
def _encode_sc(x0, x1, x2, tab):
    mesh = plsc.VectorSubcoreMesh(core_axis_name="c", subcore_axis_name="s")
    return pl.kernel(
        _body,
        out_type=jax.ShapeDtypeStruct((N_POINTS, NF), jnp.float32),
        mesh=mesh,
        compiler_params=pltpu.CompilerParams(
            needs_layout_passes=False, use_tc_tiling_on_sc=False
        ),
        scratch_types=[
            pltpu.VMEM((3, B), jnp.float32),
            pltpu.VMEM((3, 3, B), jnp.float32),
            pltpu.VMEM((3, 8 * B), jnp.int32),
            pltpu.VMEM((3, 8 * B), jnp.int32),
            pltpu.VMEM((B, NF), jnp.float32),
            pltpu.SemaphoreType.DMA,
            pltpu.SemaphoreType.DMA,
            pltpu.SemaphoreType.DMA,
        ],
    )(x0, x1, x2, tab)


def kernel(x, table):
    x = x.astype(jnp.float32)
    b0 = lax.bitcast_convert_type(
        table[:, :, 0].astype(jnp.bfloat16), jnp.uint16).astype(jnp.int32)
    b1 = lax.bitcast_convert_type(
        table[:, :, 1].astype(jnp.bfloat16), jnp.uint16).astype(jnp.int32)
    tab = ((b0 << 16) | b1).reshape(N_LEVELS * T)
    return _encode_sc(x[:, 0], x[:, 1], x[:, 2], tab)
